# Initial kernel scaffold; baseline (speedup 1.0000x reference)
import jax
import jax.numpy as jnp
from jax.experimental import pallas as pl


def kernel(x, conv1_w, conv1_s, conv1_t, conv2_w, conv2_s, conv2_t, conv3_w, conv3_s, conv3_t, cls_w1, cls_s1, cls_t1, cls_w2, cls_s2, cls_t2, cls_w3, cls_t3):
    raise NotImplementedError("write your pallas kernel here")


# speedup vs baseline: 1.7102x; 1.7102x over previous
import jax
import jax.numpy as jnp
from jax.experimental import pallas as pl
from jax.experimental.pallas import tpu as pltpu

_CH = 8
_TN = 256
_VMEM = dict(vmem_limit_bytes=48 * 1024 * 1024)


def _c1_body(c_ref, w_ref, s_ref, t_ref, o_ref):
    z = jnp.dot(c_ref[...], w_ref[...], preferred_element_type=jnp.float32)
    z = jnp.maximum(z * s_ref[...] + t_ref[...], 0.0)
    z = z.reshape(_CH * 392, 2, 64).max(axis=1)
    z = z.reshape(_CH, 14, 2, 14, 64).max(axis=2)
    o_ref[...] = z.reshape(_CH * 196, 64).astype(o_ref.dtype)


def _c2_body(x_ref, w_ref, s_ref, t_ref, o_ref):
    x = x_ref[0]
    m = _CH * 252
    z = jnp.dot(x[0:m], w_ref[0:320], preferred_element_type=jnp.float32)
    for dx in range(1, 5):
        z += jnp.dot(x[dx:dx + m], w_ref[dx * 320:(dx + 1) * 320],
                     preferred_element_type=jnp.float32)
    z = jnp.maximum(z * s_ref[...] + t_ref[...], 0.0)
    z = z.reshape(_CH, 14, 18, 64)[:, :, 2:16, :]
    z = z.reshape(_CH, 14, 7, 2, 64).max(axis=3)
    z = z.reshape(_CH, 7, 2, 7, 64).max(axis=2)
    o_ref[...] = z.reshape(_CH * 49, 64).astype(o_ref.dtype)


def _c3_body(x_ref, w_ref, s_ref, t_ref, o_ref):
    x = x_ref[0]
    m = _CH * 77
    z = jnp.dot(x[0:m], w_ref[0:320], preferred_element_type=jnp.float32)
    for dx in range(1, 5):
        z += jnp.dot(x[dx:dx + m], w_ref[dx * 320:(dx + 1) * 320],
                     preferred_element_type=jnp.float32)
    z = jnp.maximum(z * s_ref[...] + t_ref[...], 0.0)
    z = z.reshape(_CH, 7, 11, 128)[:, :, 2:9, :]
    o_ref[...] = z.reshape(_CH * 49, 128)


def _cls_body(f_ref, w1_ref, s1_ref, t1_ref, w2_ref, s2_ref, t2_ref,
              w3_ref, b3_ref, h_ref, o_ref, acc_ref):
    j = pl.program_id(1)
    h = jnp.dot(f_ref[...], w1_ref[...], preferred_element_type=jnp.float32)
    h = jnp.maximum(h * s1_ref[...] + t1_ref[...], 0.0)
    h_ref[...] = h
    part = jnp.dot(h.astype(jnp.bfloat16), w2_ref[...],
                   preferred_element_type=jnp.float32)

    @pl.when(j == 0)
    def _():
        acc_ref[...] = part

    @pl.when(j > 0)
    def _():
        acc_ref[...] += part

    @pl.when(j == pl.num_programs(1) - 1)
    def _():
        z = jnp.maximum(acc_ref[...] * s2_ref[0] + t2_ref[0], 0.0)
        o_ref[...] = jnp.dot(z.astype(jnp.bfloat16), w3_ref[...],
                             preferred_element_type=jnp.float32) + b3_ref[0]


def _dy_merge(p, n, h, w, c):
    g = n // _CH
    xp = jnp.pad(p.reshape(n, h, w, c), ((0, 0), (2, 2), (2, 2), (0, 0)))
    x5 = jnp.concatenate([xp[:, dy:dy + h, :, :] for dy in range(5)], axis=-1)
    x5 = x5.reshape(g, _CH * h * (w + 4), 5 * c)
    return jnp.pad(x5, ((0, 0), (2, 2), (0, 0)))


def kernel(x, conv1_w, conv1_s, conv1_t, conv2_w, conv2_s, conv2_t,
           conv3_w, conv3_s, conv3_t, cls_w1, cls_s1, cls_t1,
           cls_w2, cls_s2, cls_t2, cls_w3, cls_t3):
    n = x.shape[0]
    g = n // _CH

    xh = jnp.transpose(x, (0, 2, 3, 1)).astype(jnp.bfloat16)
    xp = jnp.pad(xh, ((0, 0), (2, 2), (2, 2), (0, 0)))
    cols = jnp.concatenate([xp[:, dy:dy + 28, dx:dx + 28, :]
                            for dy in range(5) for dx in range(5)], axis=-1)
    cols = jnp.pad(cols, ((0, 0), (0, 0), (0, 0), (0, 53)))
    cols = cols.reshape(n * 784, 128)

    w1 = conv1_w[:, :64]
    s1 = conv1_s[:64].reshape(1, 64)
    t1 = conv1_t[:64].reshape(1, 64)
    p1 = pl.pallas_call(
        _c1_body,
        out_shape=jax.ShapeDtypeStruct((n * 196, 64), jnp.bfloat16),
        grid=(g,),
        in_specs=[
            pl.BlockSpec((_CH * 784, 128), lambda i: (i, 0)),
            pl.BlockSpec((128, 64), lambda i: (0, 0)),
            pl.BlockSpec((1, 64), lambda i: (0, 0)),
            pl.BlockSpec((1, 64), lambda i: (0, 0)),
        ],
        out_specs=pl.BlockSpec((_CH * 196, 64), lambda i: (i, 0)),
        compiler_params=pltpu.CompilerParams(
            dimension_semantics=("parallel",), **_VMEM),
    )(cols, w1, s1, t1)

    x5_2 = _dy_merge(p1, n, 14, 14, 64)
    w2r = conv2_w[:1600, :64].reshape(5, 5, 64, 64)
    w5_2 = jnp.transpose(w2r, (1, 0, 2, 3)).reshape(1600, 64)
    s2 = conv2_s[:64].reshape(1, 64)
    t2 = conv2_t[:64].reshape(1, 64)
    p2 = pl.pallas_call(
        _c2_body,
        out_shape=jax.ShapeDtypeStruct((n * 49, 64), jnp.bfloat16),
        grid=(g,),
        in_specs=[
            pl.BlockSpec((1, _CH * 252 + 4, 320), lambda i: (i, 0, 0)),
            pl.BlockSpec((1600, 64), lambda i: (0, 0)),
            pl.BlockSpec((1, 64), lambda i: (0, 0)),
            pl.BlockSpec((1, 64), lambda i: (0, 0)),
        ],
        out_specs=pl.BlockSpec((_CH * 49, 64), lambda i: (i, 0)),
        compiler_params=pltpu.CompilerParams(
            dimension_semantics=("parallel",), **_VMEM),
    )(x5_2, w5_2, s2, t2)

    x5_3 = _dy_merge(p2, n, 7, 7, 64)
    w3r = conv3_w.reshape(5, 5, 64, 128)
    w5_3 = jnp.transpose(w3r, (1, 0, 2, 3)).reshape(1600, 128)
    s3 = conv3_s.reshape(1, 128)
    t3 = conv3_t.reshape(1, 128)
    y3 = pl.pallas_call(
        _c3_body,
        out_shape=jax.ShapeDtypeStruct((n * 49, 128), jnp.float32),
        grid=(g,),
        in_specs=[
            pl.BlockSpec((1, _CH * 77 + 4, 320), lambda i: (i, 0, 0)),
            pl.BlockSpec((1600, 128), lambda i: (0, 0)),
            pl.BlockSpec((1, 128), lambda i: (0, 0)),
            pl.BlockSpec((1, 128), lambda i: (0, 0)),
        ],
        out_specs=pl.BlockSpec((_CH * 49, 128), lambda i: (i, 0)),
        compiler_params=pltpu.CompilerParams(
            dimension_semantics=("parallel",), **_VMEM),
    )(x5_3, w5_3, s3, t3)

    feat = jnp.transpose(y3.reshape(n, 49, 128), (0, 2, 1))
    feat = feat.reshape(n, 6272).astype(jnp.bfloat16)

    nj = (cls_w1.shape[1] // 2) // _TN
    h, logits = pl.pallas_call(
        _cls_body,
        out_shape=(
            jax.ShapeDtypeStruct((n, cls_w1.shape[1]), jnp.float32),
            jax.ShapeDtypeStruct((2 * n, 128), jnp.float32),
        ),
        grid=(2, nj),
        in_specs=[
            pl.BlockSpec((n, 6272), lambda hh, j: (0, 0)),
            pl.BlockSpec((6272, _TN), lambda hh, j: (0, hh * nj + j)),
            pl.BlockSpec((1, _TN), lambda hh, j: (0, hh * nj + j)),
            pl.BlockSpec((1, _TN), lambda hh, j: (0, hh * nj + j)),
            pl.BlockSpec((_TN, 512), lambda hh, j: (hh * nj + j, 0)),
            pl.BlockSpec((1, 1, 512), lambda hh, j: (hh, 0, 0)),
            pl.BlockSpec((1, 1, 512), lambda hh, j: (hh, 0, 0)),
            pl.BlockSpec((512, 128), lambda hh, j: (hh, 0)),
            pl.BlockSpec((1, 1, 128), lambda hh, j: (hh, 0, 0)),
        ],
        out_specs=[
            pl.BlockSpec((n, _TN), lambda hh, j: (0, hh * nj + j)),
            pl.BlockSpec((n, 128), lambda hh, j: (hh, 0)),
        ],
        scratch_shapes=[pltpu.VMEM((n, 512), jnp.float32)],
        compiler_params=pltpu.CompilerParams(
            dimension_semantics=("parallel", "arbitrary"), **_VMEM),
    )(feat, cls_w1, cls_s1, cls_t1, cls_w2, cls_s2, cls_t2, cls_w3, cls_t3)

    half = cls_w1.shape[1] // 2
    return (logits[:n, :10], logits[n:2 * n, :10], h[:, :half], h[:, half:])


# speedup vs baseline: 1.7498x; 1.0232x over previous
import jax
import jax.numpy as jnp
from jax.experimental import pallas as pl
from jax.experimental.pallas import tpu as pltpu

_CH = 8
_TN = 256
_VMEM = dict(vmem_limit_bytes=48 * 1024 * 1024)


def _c1_body(c_ref, w_ref, s_ref, t_ref, o_ref):
    z = jnp.dot(c_ref[...], w_ref[...], preferred_element_type=jnp.float32)
    z = jnp.maximum(z * s_ref[...] + t_ref[...], 0.0)
    z = z.reshape(_CH * 392, 2, 64).max(axis=1)
    z = z.reshape(_CH, 14, 2, 14, 64).max(axis=2)
    o_ref[...] = z.reshape(_CH * 196, 64).astype(o_ref.dtype)


def _c2_body(x_ref, w_ref, s_ref, t_ref, o_ref):
    x = x_ref[0]
    m = _CH * 252
    z = jnp.dot(x[0:m], w_ref[0:320], preferred_element_type=jnp.float32)
    for dx in range(1, 5):
        z += jnp.dot(x[dx:dx + m], w_ref[dx * 320:(dx + 1) * 320],
                     preferred_element_type=jnp.float32)
    z = jnp.maximum(z * s_ref[...] + t_ref[...], 0.0)
    z = z.reshape(_CH, 14, 18, 64)[:, :, 2:16, :]
    z = z.reshape(_CH, 14, 7, 2, 64).max(axis=3)
    z = z.reshape(_CH, 7, 2, 7, 64).max(axis=2)
    o_ref[...] = z.reshape(_CH * 49, 64).astype(o_ref.dtype)


def _c3_body(x_ref, w_ref, s_ref, t_ref, o_ref):
    x = x_ref[0]
    m = _CH * 77
    z = jnp.dot(x[0:m], w_ref[0:320], preferred_element_type=jnp.float32)
    for dx in range(1, 5):
        z += jnp.dot(x[dx:dx + m], w_ref[dx * 320:(dx + 1) * 320],
                     preferred_element_type=jnp.float32)
    z = jnp.maximum(z * s_ref[...] + t_ref[...], 0.0)
    z = z.reshape(_CH, 7, 11, 128)[:, :, 2:9, :]
    o_ref[...] = z.reshape(_CH * 49, 128)


def _cls_body(y3_ref, w1_ref, s1_ref, t1_ref, w2_ref, s2_ref, t2_ref,
              w3_ref, b3_ref, h_ref, o_ref, feat_ref, acc_ref):
    j = pl.program_id(1)

    @pl.when(j == 0)
    def _():
        nb = y3_ref.shape[0] // 49
        f = jnp.transpose(y3_ref[...].reshape(nb, 49, 128), (0, 2, 1))
        feat_ref[...] = f.reshape(nb, 6272).astype(jnp.bfloat16)

    h = jnp.dot(feat_ref[...], w1_ref[...], preferred_element_type=jnp.float32)
    h = jnp.maximum(h * s1_ref[...] + t1_ref[...], 0.0)
    h_ref[...] = h
    part = jnp.dot(h.astype(jnp.bfloat16), w2_ref[...],
                   preferred_element_type=jnp.float32)

    @pl.when(j == 0)
    def _():
        acc_ref[...] = part

    @pl.when(j > 0)
    def _():
        acc_ref[...] += part

    @pl.when(j == pl.num_programs(1) - 1)
    def _():
        z = jnp.maximum(acc_ref[...] * s2_ref[0] + t2_ref[0], 0.0)
        o_ref[...] = jnp.dot(z.astype(jnp.bfloat16), w3_ref[...],
                             preferred_element_type=jnp.float32) + b3_ref[0]


def _dy_merge(p, n, h, w, c):
    g = n // _CH
    xp = jnp.pad(p.reshape(n, h, w, c), ((0, 0), (2, 2), (2, 2), (0, 0)))
    x5 = jnp.concatenate([xp[:, dy:dy + h, :, :] for dy in range(5)], axis=-1)
    x5 = x5.reshape(g, _CH * h * (w + 4), 5 * c)
    return jnp.pad(x5, ((0, 0), (2, 2), (0, 0)))


def kernel(x, conv1_w, conv1_s, conv1_t, conv2_w, conv2_s, conv2_t,
           conv3_w, conv3_s, conv3_t, cls_w1, cls_s1, cls_t1,
           cls_w2, cls_s2, cls_t2, cls_w3, cls_t3):
    n = x.shape[0]
    g = n // _CH

    xh = jnp.transpose(x, (0, 2, 3, 1)).astype(jnp.bfloat16)
    xp = jnp.pad(xh, ((0, 0), (2, 2), (2, 2), (0, 0)))
    cols = jnp.concatenate([xp[:, dy:dy + 28, dx:dx + 28, :]
                            for dy in range(5) for dx in range(5)], axis=-1)
    cols = jnp.pad(cols, ((0, 0), (0, 0), (0, 0), (0, 53)))
    cols = cols.reshape(n * 784, 128)

    w1 = conv1_w[:, :64]
    s1 = conv1_s[:64].reshape(1, 64)
    t1 = conv1_t[:64].reshape(1, 64)
    p1 = pl.pallas_call(
        _c1_body,
        out_shape=jax.ShapeDtypeStruct((n * 196, 64), jnp.bfloat16),
        grid=(g,),
        in_specs=[
            pl.BlockSpec((_CH * 784, 128), lambda i: (i, 0)),
            pl.BlockSpec((128, 64), lambda i: (0, 0)),
            pl.BlockSpec((1, 64), lambda i: (0, 0)),
            pl.BlockSpec((1, 64), lambda i: (0, 0)),
        ],
        out_specs=pl.BlockSpec((_CH * 196, 64), lambda i: (i, 0)),
        compiler_params=pltpu.CompilerParams(
            dimension_semantics=("parallel",), **_VMEM),
    )(cols, w1, s1, t1)

    x5_2 = _dy_merge(p1, n, 14, 14, 64)
    w2r = conv2_w[:1600, :64].reshape(5, 5, 64, 64)
    w5_2 = jnp.transpose(w2r, (1, 0, 2, 3)).reshape(1600, 64)
    s2 = conv2_s[:64].reshape(1, 64)
    t2 = conv2_t[:64].reshape(1, 64)
    p2 = pl.pallas_call(
        _c2_body,
        out_shape=jax.ShapeDtypeStruct((n * 49, 64), jnp.bfloat16),
        grid=(g,),
        in_specs=[
            pl.BlockSpec((1, _CH * 252 + 4, 320), lambda i: (i, 0, 0)),
            pl.BlockSpec((1600, 64), lambda i: (0, 0)),
            pl.BlockSpec((1, 64), lambda i: (0, 0)),
            pl.BlockSpec((1, 64), lambda i: (0, 0)),
        ],
        out_specs=pl.BlockSpec((_CH * 49, 64), lambda i: (i, 0)),
        compiler_params=pltpu.CompilerParams(
            dimension_semantics=("parallel",), **_VMEM),
    )(x5_2, w5_2, s2, t2)

    x5_3 = _dy_merge(p2, n, 7, 7, 64)
    w3r = conv3_w.reshape(5, 5, 64, 128)
    w5_3 = jnp.transpose(w3r, (1, 0, 2, 3)).reshape(1600, 128)
    s3 = conv3_s.reshape(1, 128)
    t3 = conv3_t.reshape(1, 128)
    y3 = pl.pallas_call(
        _c3_body,
        out_shape=jax.ShapeDtypeStruct((n * 49, 128), jnp.float32),
        grid=(g,),
        in_specs=[
            pl.BlockSpec((1, _CH * 77 + 4, 320), lambda i: (i, 0, 0)),
            pl.BlockSpec((1600, 128), lambda i: (0, 0)),
            pl.BlockSpec((1, 128), lambda i: (0, 0)),
            pl.BlockSpec((1, 128), lambda i: (0, 0)),
        ],
        out_specs=pl.BlockSpec((_CH * 49, 128), lambda i: (i, 0)),
        compiler_params=pltpu.CompilerParams(
            dimension_semantics=("parallel",), **_VMEM),
    )(x5_3, w5_3, s3, t3)

    nj = (cls_w1.shape[1] // 2) // _TN
    h, logits = pl.pallas_call(
        _cls_body,
        out_shape=(
            jax.ShapeDtypeStruct((n, cls_w1.shape[1]), jnp.float32),
            jax.ShapeDtypeStruct((2 * n, 128), jnp.float32),
        ),
        grid=(2, nj),
        in_specs=[
            pl.BlockSpec((n * 49, 128), lambda hh, j: (0, 0)),
            pl.BlockSpec((6272, _TN), lambda hh, j: (0, hh * nj + j)),
            pl.BlockSpec((1, _TN), lambda hh, j: (0, hh * nj + j)),
            pl.BlockSpec((1, _TN), lambda hh, j: (0, hh * nj + j)),
            pl.BlockSpec((_TN, 512), lambda hh, j: (hh * nj + j, 0)),
            pl.BlockSpec((1, 1, 512), lambda hh, j: (hh, 0, 0)),
            pl.BlockSpec((1, 1, 512), lambda hh, j: (hh, 0, 0)),
            pl.BlockSpec((512, 128), lambda hh, j: (hh, 0)),
            pl.BlockSpec((1, 1, 128), lambda hh, j: (hh, 0, 0)),
        ],
        out_specs=[
            pl.BlockSpec((n, _TN), lambda hh, j: (0, hh * nj + j)),
            pl.BlockSpec((n, 128), lambda hh, j: (hh, 0)),
        ],
        scratch_shapes=[pltpu.VMEM((n, 6272), jnp.bfloat16),
                        pltpu.VMEM((n, 512), jnp.float32)],
        compiler_params=pltpu.CompilerParams(
            dimension_semantics=("parallel", "arbitrary"), **_VMEM),
    )(y3, cls_w1, cls_s1, cls_t1, cls_w2, cls_s2, cls_t2, cls_w3, cls_t3)

    half = cls_w1.shape[1] // 2
    return (logits[:n, :10], logits[n:2 * n, :10], h[:, :half], h[:, half:])


# speedup vs baseline: 1.7512x; 1.0008x over previous
import jax
import jax.numpy as jnp
from jax.experimental import pallas as pl
from jax.experimental.pallas import tpu as pltpu

_CH = 8
_TN = 256
_VMEM = dict(vmem_limit_bytes=48 * 1024 * 1024)


def _c1_body(c_ref, w_ref, s_ref, t_ref, o_ref):
    z = jnp.dot(c_ref[...], w_ref[...], preferred_element_type=jnp.float32)
    z = jnp.maximum(z * s_ref[...] + t_ref[...], 0.0)
    z = z.reshape(_CH * 392, 2, 64).max(axis=1)
    z = z.reshape(_CH, 14, 2, 14, 64).max(axis=2)
    o_ref[...] = z.reshape(_CH * 196, 64).astype(o_ref.dtype)


def _c2_body(x_ref, w_ref, s_ref, t_ref, o_ref):
    x = x_ref[0]
    m = _CH * 252
    z = jnp.dot(x[0:m], w_ref[0:320], preferred_element_type=jnp.float32)
    for dx in range(1, 5):
        z += jnp.dot(x[dx:dx + m], w_ref[dx * 320:(dx + 1) * 320],
                     preferred_element_type=jnp.float32)
    z = jnp.maximum(z * s_ref[...] + t_ref[...], 0.0)
    z = z.reshape(_CH, 14, 18, 64)[:, :, 2:16, :]
    z = z.reshape(_CH, 14, 7, 2, 64).max(axis=3)
    z = z.reshape(_CH, 7, 2, 7, 64).max(axis=2)
    o_ref[...] = z.reshape(_CH * 49, 64).astype(o_ref.dtype)


def _c3_body(x_ref, w_ref, s_ref, t_ref, o_ref):
    x = x_ref[0]
    m = _CH * 77
    z = jnp.dot(x[0:m], w_ref[0:320], preferred_element_type=jnp.float32)
    for dx in range(1, 5):
        z += jnp.dot(x[dx:dx + m], w_ref[dx * 320:(dx + 1) * 320],
                     preferred_element_type=jnp.float32)
    z = jnp.maximum(z * s_ref[...] + t_ref[...], 0.0)
    z = z.reshape(_CH, 7, 11, 128)[:, :, 2:9, :]
    o_ref[...] = z.reshape(_CH * 49, 128)


def _cls_body(y3_ref, w1_ref, s1_ref, t1_ref, w2_ref, s2_ref, t2_ref,
              w3_ref, b3_ref, h_ref, o_ref, feat_ref, acc_ref):
    j = pl.program_id(1)

    @pl.when(j == 0)
    def _():
        nb = y3_ref.shape[0] // 49
        f = jnp.transpose(y3_ref[...].reshape(nb, 49, 128), (0, 2, 1))
        feat_ref[...] = f.reshape(nb, 6272).astype(jnp.bfloat16)

    h = jnp.dot(feat_ref[...], w1_ref[...], preferred_element_type=jnp.float32)
    h = jnp.maximum(h * s1_ref[...] + t1_ref[...], 0.0)
    h_ref[...] = h
    part = jnp.dot(h.astype(jnp.bfloat16), w2_ref[...],
                   preferred_element_type=jnp.float32)

    @pl.when(j == 0)
    def _():
        acc_ref[...] = part

    @pl.when(j > 0)
    def _():
        acc_ref[...] += part

    @pl.when(j == pl.num_programs(1) - 1)
    def _():
        z = jnp.maximum(acc_ref[...] * s2_ref[0] + t2_ref[0], 0.0)
        o_ref[...] = jnp.dot(z.astype(jnp.bfloat16), w3_ref[...],
                             preferred_element_type=jnp.float32) + b3_ref[0]


def _dy_merge(p, n, h, w, c):
    g = n // _CH
    xp = jnp.pad(p.reshape(n, h, w, c), ((0, 0), (2, 2), (2, 2), (0, 0)))
    x5 = jnp.concatenate([xp[:, dy:dy + h, :, :] for dy in range(5)], axis=-1)
    x5 = x5.reshape(g, _CH * h * (w + 4), 5 * c)
    return jnp.pad(x5, ((0, 0), (2, 2), (0, 0)))


def kernel(x, conv1_w, conv1_s, conv1_t, conv2_w, conv2_s, conv2_t,
           conv3_w, conv3_s, conv3_t, cls_w1, cls_s1, cls_t1,
           cls_w2, cls_s2, cls_t2, cls_w3, cls_t3):
    n = x.shape[0]
    g = n // _CH

    xh = jnp.transpose(x, (0, 2, 3, 1)).astype(jnp.bfloat16)
    xp = jnp.pad(xh, ((0, 0), (2, 2), (2, 2), (0, 0)))
    cols = jnp.concatenate([xp[:, dy:dy + 28, dx:dx + 28, :]
                            for dy in range(5) for dx in range(5)], axis=-1)
    cols = jnp.pad(cols, ((0, 0), (0, 0), (0, 0), (0, 53)))
    cols = cols.reshape(n * 784, 128)

    w1 = conv1_w[:, :64]
    s1 = conv1_s[:64].reshape(1, 64)
    t1 = conv1_t[:64].reshape(1, 64)
    p1 = pl.pallas_call(
        _c1_body,
        out_shape=jax.ShapeDtypeStruct((n * 196, 64), jnp.bfloat16),
        grid=(g,),
        in_specs=[
            pl.BlockSpec((_CH * 784, 128), lambda i: (i, 0)),
            pl.BlockSpec((128, 64), lambda i: (0, 0)),
            pl.BlockSpec((1, 64), lambda i: (0, 0)),
            pl.BlockSpec((1, 64), lambda i: (0, 0)),
        ],
        out_specs=pl.BlockSpec((_CH * 196, 64), lambda i: (i, 0)),
        compiler_params=pltpu.CompilerParams(
            dimension_semantics=("parallel",), **_VMEM),
    )(cols, w1, s1, t1)

    x5_2 = _dy_merge(p1, n, 14, 14, 64)
    w2r = conv2_w[:1600, :64].reshape(5, 5, 64, 64)
    w5_2 = jnp.transpose(w2r, (1, 0, 2, 3)).reshape(1600, 64)
    s2 = conv2_s[:64].reshape(1, 64)
    t2 = conv2_t[:64].reshape(1, 64)
    p2 = pl.pallas_call(
        _c2_body,
        out_shape=jax.ShapeDtypeStruct((n * 49, 64), jnp.bfloat16),
        grid=(g,),
        in_specs=[
            pl.BlockSpec((1, _CH * 252 + 4, 320), lambda i: (i, 0, 0)),
            pl.BlockSpec((1600, 64), lambda i: (0, 0)),
            pl.BlockSpec((1, 64), lambda i: (0, 0)),
            pl.BlockSpec((1, 64), lambda i: (0, 0)),
        ],
        out_specs=pl.BlockSpec((_CH * 49, 64), lambda i: (i, 0)),
        compiler_params=pltpu.CompilerParams(
            dimension_semantics=("parallel",), **_VMEM),
    )(x5_2, w5_2, s2, t2)

    x5_3 = _dy_merge(p2, n, 7, 7, 64)
    w3r = conv3_w.reshape(5, 5, 64, 128)
    w5_3 = jnp.transpose(w3r, (1, 0, 2, 3)).reshape(1600, 128)
    s3 = conv3_s.reshape(1, 128)
    t3 = conv3_t.reshape(1, 128)
    y3 = pl.pallas_call(
        _c3_body,
        out_shape=jax.ShapeDtypeStruct((n * 49, 128), jnp.float32),
        grid=(g,),
        in_specs=[
            pl.BlockSpec((1, _CH * 77 + 4, 320), lambda i: (i, 0, 0)),
            pl.BlockSpec((1600, 128), lambda i: (0, 0)),
            pl.BlockSpec((1, 128), lambda i: (0, 0)),
            pl.BlockSpec((1, 128), lambda i: (0, 0)),
        ],
        out_specs=pl.BlockSpec((_CH * 49, 128), lambda i: (i, 0)),
        compiler_params=pltpu.CompilerParams(
            dimension_semantics=("parallel",), **_VMEM),
    )(x5_3, w5_3, s3, t3)

    nj = (cls_w1.shape[1] // 2) // _TN
    h, logits = pl.pallas_call(
        _cls_body,
        out_shape=(
            jax.ShapeDtypeStruct((2 * n, cls_w1.shape[1] // 2), jnp.float32),
            jax.ShapeDtypeStruct((2 * n, 128), jnp.float32),
        ),
        grid=(2, nj),
        in_specs=[
            pl.BlockSpec((n * 49, 128), lambda hh, j: (0, 0)),
            pl.BlockSpec((6272, _TN), lambda hh, j: (0, hh * nj + j)),
            pl.BlockSpec((1, _TN), lambda hh, j: (0, hh * nj + j)),
            pl.BlockSpec((1, _TN), lambda hh, j: (0, hh * nj + j)),
            pl.BlockSpec((_TN, 512), lambda hh, j: (hh * nj + j, 0)),
            pl.BlockSpec((1, 1, 512), lambda hh, j: (hh, 0, 0)),
            pl.BlockSpec((1, 1, 512), lambda hh, j: (hh, 0, 0)),
            pl.BlockSpec((512, 128), lambda hh, j: (hh, 0)),
            pl.BlockSpec((1, 1, 128), lambda hh, j: (hh, 0, 0)),
        ],
        out_specs=[
            pl.BlockSpec((n, _TN), lambda hh, j: (hh, j)),
            pl.BlockSpec((n, 128), lambda hh, j: (hh, 0)),
        ],
        scratch_shapes=[pltpu.VMEM((n, 6272), jnp.bfloat16),
                        pltpu.VMEM((n, 512), jnp.float32)],
        compiler_params=pltpu.CompilerParams(
            dimension_semantics=("parallel", "arbitrary"), **_VMEM),
    )(y3, cls_w1, cls_s1, cls_t1, cls_w2, cls_s2, cls_t2, cls_w3, cls_t3)

    return (logits[:n, :10], logits[n:2 * n, :10], h[:n], h[n:2 * n])
